# baseline (speedup 1.0000x reference)
import functools
import jax
import jax.numpy as jnp
from jax import lax
from jax.experimental import pallas as pl
from jax.experimental.pallas import tpu as pltpu
from jax.experimental.pallas import tpu_sc as plsc

N_TILE = 3072


def _mq_tc_kernel(q_ref, wt_ref, idx_ref, oh_ref, perp_ref, counts_ref):
    i = pl.program_id(0)
    nsteps = pl.num_programs(0)
    q = q_ref[:]
    wt = wt_ref[:]
    sim = jnp.dot(q, wt, preferred_element_type=jnp.float32)
    l2q = jnp.sum(q * q, axis=1, keepdims=True)
    l2k = jnp.sum(wt * wt, axis=0, keepdims=True)
    dist = (l2q + l2k) - 2.0 * sim
    idx = jnp.argmin(dist, axis=1).astype(jnp.int32)
    idx_ref[:] = idx[:, None]
    iota = jax.lax.broadcasted_iota(jnp.int32, dist.shape, 1)
    oh = (iota == idx[:, None]).astype(jnp.float32)
    oh_ref[:] = oh

    tile_counts = jnp.sum(oh, axis=0, keepdims=True)

    @pl.when(i == 0)
    def _():
        counts_ref[:] = tile_counts

    @pl.when(i > 0)
    def _():
        counts_ref[:] = counts_ref[:] + tile_counts

    @pl.when(i == nsteps - 1)
    def _():
        n_total = nsteps * q.shape[0]
        z_mean = counts_ref[:] / n_total
        ent = jnp.sum(z_mean * jnp.log(z_mean + 1e-10), axis=1, keepdims=True)
        perp_ref[:] = jnp.exp(-ent)


def _sc_gather(W, idx_flat, N, C):
    info = plsc.get_sparse_core_info()
    NC, NS = info.num_cores, info.num_subcores
    NW = NC * NS
    b_per_w = N // NW
    n_chunk = 3
    chunk = b_per_w // n_chunk
    mesh = plsc.VectorSubcoreMesh(core_axis_name="c", subcore_axis_name="s")

    @functools.partial(
        pl.kernel,
        mesh=mesh,
        out_type=jax.ShapeDtypeStruct((N, C), jnp.float32),
        scratch_types=[
            pltpu.VMEM((n_chunk, chunk), jnp.int32),
            pltpu.VMEM((b_per_w, C), jnp.float32),
            pltpu.SemaphoreType.DMA,
        ],
    )
    def gather_kernel(table_hbm, idx_hbm, out_hbm, idx_v, rows_v, sem):
        wid = lax.axis_index("s") * NC + lax.axis_index("c")
        base = wid * b_per_w
        for j in range(n_chunk):
            pltpu.sync_copy(idx_hbm.at[pl.ds(base + j * chunk, chunk)],
                            idx_v.at[j])
        for j in range(n_chunk):
            pltpu.async_copy(
                table_hbm.at[idx_v.at[j]],
                rows_v.at[pl.ds(j * chunk, chunk)],
                sem,
            ).wait()
        pltpu.sync_copy(rows_v, out_hbm.at[pl.ds(base, b_per_w)])

    return gather_kernel(W, idx_flat)


def kernel(x, W):
    b, n, c = x.shape
    q = x.reshape(b * n, c)
    N = b * n
    K = W.shape[0]
    wt = W.T
    nsteps = N // N_TILE

    idx, oh, perp = pl.pallas_call(
        _mq_tc_kernel,
        grid=(nsteps,),
        in_specs=[
            pl.BlockSpec((N_TILE, c), lambda i: (i, 0)),
            pl.BlockSpec((c, K), lambda i: (0, 0)),
        ],
        out_specs=[
            pl.BlockSpec((N_TILE, 1), lambda i: (i, 0)),
            pl.BlockSpec((N_TILE, K), lambda i: (i, 0)),
            pl.BlockSpec((1, 1), lambda i: (0, 0)),
        ],
        out_shape=[
            jax.ShapeDtypeStruct((N, 1), jnp.int32),
            jax.ShapeDtypeStruct((N, K), jnp.float32),
            jax.ShapeDtypeStruct((1, 1), jnp.float32),
        ],
        scratch_shapes=[pltpu.VMEM((1, K), jnp.float32)],
    )(q, wt)

    idx_flat = idx.reshape(N)
    emb = _sc_gather(W, idx_flat, N, c)

    embedding = emb.reshape(b, n, c)
    indices = idx.reshape(b, n)
    onehot = oh.reshape(b, n, K)
    perplexity = perp[0, 0]
    return embedding, indices, onehot, perplexity


# speedup vs baseline: 1.0368x; 1.0368x over previous
import functools
import jax
import jax.numpy as jnp
from jax import lax
from jax.experimental import pallas as pl
from jax.experimental.pallas import tpu as pltpu
from jax.experimental.pallas import tpu_sc as plsc

N_TILE = 3072


def _mq_tc_kernel(q_ref, wt_ref, idx_ref, oh_ref, perp_ref, counts_ref):
    i = pl.program_id(0)
    nsteps = pl.num_programs(0)
    q = q_ref[:]
    wt = wt_ref[:]
    sim = jnp.dot(q, wt, preferred_element_type=jnp.float32)
    l2q = jnp.sum(q * q, axis=1, keepdims=True)
    l2k = jnp.sum(wt * wt, axis=0, keepdims=True)
    dist = (l2q + l2k) - 2.0 * sim
    idx = jnp.argmin(dist, axis=1).astype(jnp.int32)
    idx_ref[:] = idx[:, None]
    iota = jax.lax.broadcasted_iota(jnp.int32, dist.shape, 1)
    oh = (iota == idx[:, None]).astype(jnp.float32)
    oh_ref[:] = oh

    tile_counts = jnp.sum(oh, axis=0, keepdims=True)

    @pl.when(i == 0)
    def _():
        counts_ref[:] = tile_counts

    @pl.when(i > 0)
    def _():
        counts_ref[:] = counts_ref[:] + tile_counts

    @pl.when(i == nsteps - 1)
    def _():
        n_total = nsteps * q.shape[0]
        z_mean = counts_ref[:] / n_total
        ent = jnp.sum(z_mean * jnp.log(z_mean + 1e-10), axis=1, keepdims=True)
        perp_ref[:] = jnp.exp(-ent)


def _sc_gather(W, idx_flat, N, C):
    info = plsc.get_sparse_core_info()
    NC, NS = info.num_cores, info.num_subcores
    NW = NC * NS
    b_per_w = N // NW
    n_chunk = 3
    chunk = b_per_w // n_chunk
    mesh = plsc.VectorSubcoreMesh(core_axis_name="c", subcore_axis_name="s")

    @functools.partial(
        pl.kernel,
        mesh=mesh,
        out_type=jax.ShapeDtypeStruct((N, C), jnp.float32),
    scratch_types=[
            pltpu.VMEM((n_chunk, chunk), jnp.int32),
            pltpu.VMEM((b_per_w, C), jnp.float32),
            pltpu.SemaphoreType.DMA,
            pltpu.SemaphoreType.DMA,
        ],
    )
    def gather_kernel(table_hbm, idx_hbm, out_hbm, idx_v, rows_v, isem, gsem):
        wid = lax.axis_index("s") * NC + lax.axis_index("c")
        base = wid * b_per_w
        copies = [
            pltpu.async_copy(idx_hbm.at[pl.ds(base + j * chunk, chunk)],
                             idx_v.at[j], isem)
            for j in range(n_chunk)
        ]
        for cp in copies:
            cp.wait()
        gathers = [
            pltpu.async_copy(table_hbm.at[idx_v.at[j]],
                             rows_v.at[pl.ds(j * chunk, chunk)], gsem)
            for j in range(n_chunk)
        ]
        for g in gathers:
            g.wait()
        pltpu.sync_copy(rows_v, out_hbm.at[pl.ds(base, b_per_w)])

    return gather_kernel(W, idx_flat)


def kernel(x, W):
    b, n, c = x.shape
    q = x.reshape(b * n, c)
    N = b * n
    K = W.shape[0]
    wt = W.T
    nsteps = N // N_TILE

    idx, oh, perp = pl.pallas_call(
        _mq_tc_kernel,
        grid=(nsteps,),
        in_specs=[
            pl.BlockSpec((N_TILE, c), lambda i: (i, 0)),
            pl.BlockSpec((c, K), lambda i: (0, 0)),
        ],
        out_specs=[
            pl.BlockSpec((N_TILE, 1), lambda i: (i, 0)),
            pl.BlockSpec((N_TILE, K), lambda i: (i, 0)),
            pl.BlockSpec((1, 1), lambda i: (0, 0)),
        ],
        out_shape=[
            jax.ShapeDtypeStruct((N, 1), jnp.int32),
            jax.ShapeDtypeStruct((N, K), jnp.float32),
            jax.ShapeDtypeStruct((1, 1), jnp.float32),
        ],
        scratch_shapes=[pltpu.VMEM((1, K), jnp.float32)],
    )(q, wt)

    idx_flat = idx.reshape(N)
    emb = _sc_gather(W, idx_flat, N, c)

    embedding = emb.reshape(b, n, c)
    indices = idx.reshape(b, n)
    onehot = oh.reshape(b, n, K)
    perplexity = perp[0, 0]
    return embedding, indices, onehot, perplexity


# speedup vs baseline: 1.7291x; 1.6677x over previous
import jax
import jax.numpy as jnp
from jax.experimental import pallas as pl
from jax.experimental.pallas import tpu as pltpu

N_TILE = 3072


def _mq_kernel(q_ref, w_ref, wt_ref, emb_ref, idx_ref, oh_ref, perp_ref,
               counts_ref):
    i = pl.program_id(0)
    nsteps = pl.num_programs(0)
    q = q_ref[:]
    wt = wt_ref[:]
    sim = jnp.dot(q, wt, preferred_element_type=jnp.float32)
    l2q = jnp.sum(q * q, axis=1, keepdims=True)
    l2k = jnp.sum(wt * wt, axis=0, keepdims=True)
    dist = (l2q + l2k) - 2.0 * sim
    idx = jnp.argmin(dist, axis=1).astype(jnp.int32)
    idx_ref[:] = idx[:, None]
    iota = jax.lax.broadcasted_iota(jnp.int32, dist.shape, 1)
    oh = (iota == idx[:, None]).astype(jnp.float32)
    oh_ref[:] = oh
    emb_ref[:] = jnp.dot(oh.astype(jnp.bfloat16),
                         w_ref[:].astype(jnp.bfloat16),
                         preferred_element_type=jnp.float32)

    tile_counts = jnp.sum(oh, axis=0, keepdims=True)

    @pl.when(i == 0)
    def _():
        counts_ref[:] = tile_counts

    @pl.when(i > 0)
    def _():
        counts_ref[:] = counts_ref[:] + tile_counts

    @pl.when(i == nsteps - 1)
    def _():
        n_total = nsteps * q.shape[0]
        z_mean = counts_ref[:] / n_total
        ent = jnp.sum(z_mean * jnp.log(z_mean + 1e-10), axis=1, keepdims=True)
        perp_ref[:] = jnp.exp(-ent)


def kernel(x, W):
    b, n, c = x.shape
    q = x.reshape(b * n, c)
    N = b * n
    K = W.shape[0]
    wt = W.T
    nsteps = N // N_TILE

    emb, idx, oh, perp = pl.pallas_call(
        _mq_kernel,
        grid=(nsteps,),
        in_specs=[
            pl.BlockSpec((N_TILE, c), lambda i: (i, 0)),
            pl.BlockSpec((K, c), lambda i: (0, 0)),
            pl.BlockSpec((c, K), lambda i: (0, 0)),
        ],
        out_specs=[
            pl.BlockSpec((N_TILE, c), lambda i: (i, 0)),
            pl.BlockSpec((N_TILE, 1), lambda i: (i, 0)),
            pl.BlockSpec((N_TILE, K), lambda i: (i, 0)),
            pl.BlockSpec((1, 1), lambda i: (0, 0)),
        ],
        out_shape=[
            jax.ShapeDtypeStruct((N, c), jnp.float32),
            jax.ShapeDtypeStruct((N, 1), jnp.int32),
            jax.ShapeDtypeStruct((N, K), jnp.float32),
            jax.ShapeDtypeStruct((1, 1), jnp.float32),
        ],
        scratch_shapes=[pltpu.VMEM((1, K), jnp.float32)],
    )(q, W, wt)

    embedding = emb.reshape(b, n, c)
    indices = idx.reshape(b, n)
    onehot = oh.reshape(b, n, K)
    perplexity = perp[0, 0]
    return embedding, indices, onehot, perplexity


# speedup vs baseline: 1.8694x; 1.0812x over previous
import jax
import jax.numpy as jnp
from jax.experimental import pallas as pl
from jax.experimental.pallas import tpu as pltpu

N_TILE = 1536


def _mq_kernel(q_ref, w_ref, wt_ref, emb_ref, idx_ref, oh_ref, perp_ref,
               counts_ref):
    i = pl.program_id(0)
    nsteps = pl.num_programs(0)
    q = q_ref[:]
    wt = wt_ref[:]
    sim = jnp.dot(q, wt, preferred_element_type=jnp.float32)
    l2q = jnp.sum(q * q, axis=1, keepdims=True)
    l2k = jnp.sum(wt * wt, axis=0, keepdims=True)
    dist = (l2q + l2k) - 2.0 * sim
    idx = jnp.argmin(dist, axis=1).astype(jnp.int32)
    idx_ref[:] = idx[:, None]
    iota = jax.lax.broadcasted_iota(jnp.int32, dist.shape, 1)
    oh = (iota == idx[:, None]).astype(jnp.float32)
    oh_ref[:] = oh
    emb_ref[:] = jnp.dot(oh, w_ref[:], preferred_element_type=jnp.float32)

    tile_counts = jnp.sum(oh, axis=0, keepdims=True)

    @pl.when(i == 0)
    def _():
        counts_ref[:] = tile_counts

    @pl.when(i > 0)
    def _():
        counts_ref[:] = counts_ref[:] + tile_counts

    @pl.when(i == nsteps - 1)
    def _():
        n_total = nsteps * q.shape[0]
        z_mean = counts_ref[:] / n_total
        ent = jnp.sum(z_mean * jnp.log(z_mean + 1e-10), axis=1, keepdims=True)
        perp_ref[:] = jnp.exp(-ent)


def kernel(x, W):
    b, n, c = x.shape
    q = x.reshape(b * n, c)
    N = b * n
    K = W.shape[0]
    wt = W.T
    nsteps = N // N_TILE

    emb, idx, oh, perp = pl.pallas_call(
        _mq_kernel,
        grid=(nsteps,),
        in_specs=[
            pl.BlockSpec((N_TILE, c), lambda i: (i, 0)),
            pl.BlockSpec((K, c), lambda i: (0, 0)),
            pl.BlockSpec((c, K), lambda i: (0, 0)),
        ],
        out_specs=[
            pl.BlockSpec((N_TILE, c), lambda i: (i, 0)),
            pl.BlockSpec((N_TILE, 1), lambda i: (i, 0)),
            pl.BlockSpec((N_TILE, K), lambda i: (i, 0)),
            pl.BlockSpec((1, 1), lambda i: (0, 0)),
        ],
        out_shape=[
            jax.ShapeDtypeStruct((N, c), jnp.float32),
            jax.ShapeDtypeStruct((N, 1), jnp.int32),
            jax.ShapeDtypeStruct((N, K), jnp.float32),
            jax.ShapeDtypeStruct((1, 1), jnp.float32),
        ],
        scratch_shapes=[pltpu.VMEM((1, K), jnp.float32)],
    )(q, W, wt)

    embedding = emb.reshape(b, n, c)
    indices = idx.reshape(b, n)
    onehot = oh.reshape(b, n, K)
    perplexity = perp[0, 0]
    return embedding, indices, onehot, perplexity


# speedup vs baseline: 1.9703x; 1.0540x over previous
import jax
import jax.numpy as jnp
from jax.experimental import pallas as pl
from jax.experimental.pallas import tpu as pltpu

N_TILE = 3072


def _mq_kernel(q_ref, w_ref, wt_ref, emb_ref, idx_ref, oh_ref, perp_ref,
               idx_s, counts_ref):
    i = pl.program_id(0)
    nsteps = pl.num_programs(0) - 1
    K = w_ref.shape[0]

    @pl.when(i > 0)
    def _():
        idx_prev = idx_s[:]
        iota = jax.lax.broadcasted_iota(jnp.int32, (idx_prev.shape[0], K), 1)
        oh = (iota == idx_prev).astype(jnp.float32)
        oh_ref[:] = oh
        emb_ref[:] = jnp.dot(oh, w_ref[:], preferred_element_type=jnp.float32)
        tile_counts = jnp.sum(oh, axis=0, keepdims=True)

        @pl.when(i == 1)
        def _():
            counts_ref[:] = tile_counts

        @pl.when(i > 1)
        def _():
            counts_ref[:] = counts_ref[:] + tile_counts

    @pl.when(i < nsteps)
    def _():
        q = q_ref[:]
        wt = wt_ref[:]
        sim = jnp.dot(q, wt, preferred_element_type=jnp.float32)
        l2q = jnp.sum(q * q, axis=1, keepdims=True)
        l2k = jnp.sum(wt * wt, axis=0, keepdims=True)
        dist = (l2q + l2k) - 2.0 * sim
        idx = jnp.argmin(dist, axis=1).astype(jnp.int32)
        idx_ref[:] = idx[:, None]
        idx_s[:] = idx[:, None]

    @pl.when(i == nsteps)
    def _():
        n_total = nsteps * q_ref.shape[0]
        z_mean = counts_ref[:] / n_total
        ent = jnp.sum(z_mean * jnp.log(z_mean + 1e-10), axis=1, keepdims=True)
        perp_ref[:] = jnp.exp(-ent)


def kernel(x, W):
    b, n, c = x.shape
    q = x.reshape(b * n, c)
    N = b * n
    K = W.shape[0]
    wt = W.T
    nsteps = N // N_TILE

    def cur(i):
        return jnp.minimum(i, nsteps - 1)

    def prev(i):
        return jnp.maximum(i - 1, 0)

    emb, idx, oh, perp = pl.pallas_call(
        _mq_kernel,
        grid=(nsteps + 1,),
        in_specs=[
            pl.BlockSpec((N_TILE, c), lambda i: (cur(i), 0)),
            pl.BlockSpec((K, c), lambda i: (0, 0)),
            pl.BlockSpec((c, K), lambda i: (0, 0)),
        ],
        out_specs=[
            pl.BlockSpec((N_TILE, c), lambda i: (prev(i), 0)),
            pl.BlockSpec((N_TILE, 1), lambda i: (cur(i), 0)),
            pl.BlockSpec((N_TILE, K), lambda i: (prev(i), 0)),
            pl.BlockSpec((1, 1), lambda i: (0, 0)),
        ],
        out_shape=[
            jax.ShapeDtypeStruct((N, c), jnp.float32),
            jax.ShapeDtypeStruct((N, 1), jnp.int32),
            jax.ShapeDtypeStruct((N, K), jnp.float32),
            jax.ShapeDtypeStruct((1, 1), jnp.float32),
        ],
        scratch_shapes=[
            pltpu.VMEM((N_TILE, 1), jnp.int32),
            pltpu.VMEM((1, K), jnp.float32),
        ],
    )(q, W, wt)

    embedding = emb.reshape(b, n, c)
    indices = idx.reshape(b, n)
    onehot = oh.reshape(b, n, K)
    perplexity = perp[0, 0]
    return embedding, indices, onehot, perplexity


# speedup vs baseline: 2.0955x; 1.0636x over previous
import jax
import jax.numpy as jnp
from jax.experimental import pallas as pl
from jax.experimental.pallas import tpu as pltpu

N_TILE = 1536


def _mq_kernel(q_ref, w_ref, wt_ref, emb_ref, idx_ref, oh_ref, perp_ref,
               idx_s, counts_ref):
    i = pl.program_id(0)
    nsteps = pl.num_programs(0) - 1
    K = w_ref.shape[0]

    @pl.when(i > 0)
    def _():
        idx_prev = idx_s[:]
        iota = jax.lax.broadcasted_iota(jnp.int32, (idx_prev.shape[0], K), 1)
        oh = (iota == idx_prev).astype(jnp.float32)
        oh_ref[:] = oh
        emb_ref[:] = jnp.dot(oh, w_ref[:], preferred_element_type=jnp.float32)
        tile_counts = jnp.sum(oh, axis=0, keepdims=True)

        @pl.when(i == 1)
        def _():
            counts_ref[:] = tile_counts

        @pl.when(i > 1)
        def _():
            counts_ref[:] = counts_ref[:] + tile_counts

    @pl.when(i < nsteps)
    def _():
        q = q_ref[:]
        wt = wt_ref[:]
        sim = jnp.dot(q, wt, preferred_element_type=jnp.float32)
        l2q = jnp.sum(q * q, axis=1, keepdims=True)
        l2k = jnp.sum(wt * wt, axis=0, keepdims=True)
        dist = (l2q + l2k) - 2.0 * sim
        idx = jnp.argmin(dist, axis=1).astype(jnp.int32)
        idx_ref[:] = idx[:, None]
        idx_s[:] = idx[:, None]

    @pl.when(i == nsteps)
    def _():
        n_total = nsteps * q_ref.shape[0]
        z_mean = counts_ref[:] / n_total
        ent = jnp.sum(z_mean * jnp.log(z_mean + 1e-10), axis=1, keepdims=True)
        perp_ref[:] = jnp.exp(-ent)


def kernel(x, W):
    b, n, c = x.shape
    q = x.reshape(b * n, c)
    N = b * n
    K = W.shape[0]
    wt = W.T
    nsteps = N // N_TILE

    def cur(i):
        return jnp.minimum(i, nsteps - 1)

    def prev(i):
        return jnp.maximum(i - 1, 0)

    emb, idx, oh, perp = pl.pallas_call(
        _mq_kernel,
        grid=(nsteps + 1,),
        in_specs=[
            pl.BlockSpec((N_TILE, c), lambda i: (cur(i), 0)),
            pl.BlockSpec((K, c), lambda i: (0, 0)),
            pl.BlockSpec((c, K), lambda i: (0, 0)),
        ],
        out_specs=[
            pl.BlockSpec((N_TILE, c), lambda i: (prev(i), 0)),
            pl.BlockSpec((N_TILE, 1), lambda i: (cur(i), 0)),
            pl.BlockSpec((N_TILE, K), lambda i: (prev(i), 0)),
            pl.BlockSpec((1, 1), lambda i: (0, 0)),
        ],
        out_shape=[
            jax.ShapeDtypeStruct((N, c), jnp.float32),
            jax.ShapeDtypeStruct((N, 1), jnp.int32),
            jax.ShapeDtypeStruct((N, K), jnp.float32),
            jax.ShapeDtypeStruct((1, 1), jnp.float32),
        ],
        scratch_shapes=[
            pltpu.VMEM((N_TILE, 1), jnp.int32),
            pltpu.VMEM((1, K), jnp.float32),
        ],
    )(q, W, wt)

    embedding = emb.reshape(b, n, c)
    indices = idx.reshape(b, n)
    onehot = oh.reshape(b, n, K)
    perplexity = perp[0, 0]
    return embedding, indices, onehot, perplexity
